# baseline (speedup 1.0000x reference)
import functools

import jax
import jax.numpy as jnp
from jax import lax
from jax.experimental import pallas as pl
from jax.experimental.pallas import tpu as pltpu
from jax.experimental.pallas import tpu_sc as plsc

V = 100000
E = 768
H = 50
S = 30
C = 10
B = 16384
DP = 64
CP = 128

NIDX = B * S
NC, NS = 2, 16
NW = NC * NS
GSIZE = 128

NSPLIT = 1
BCH = B // NSPLIT
NIDXC = BCH * S
ROWS_PER_W = NIDXC // NW
GPC = 10
CHUNK = GSIZE * GPC
NCHUNK = ROWS_PER_W // CHUNK
DPH = DP // 2


_PROJ_RB = 1000


def _proj_body(emb_ref, w1t_ref, b1_ref, out_ref):
    y = (
        jnp.dot(emb_ref[...], w1t_ref[...], preferred_element_type=jnp.float32)
        + b1_ref[...]
    ).astype(jnp.bfloat16)
    lo = jax.lax.convert_element_type(
        jax.lax.bitcast_convert_type(y[:, :DPH], jnp.uint16), jnp.uint32
    )
    hi = jax.lax.convert_element_type(
        jax.lax.bitcast_convert_type(y[:, DPH:], jnp.uint16), jnp.uint32
    )
    out_ref[...] = lo | (hi << 16)


def _project_table(emb, w1t, b1p):
    return pl.pallas_call(
        _proj_body,
        grid=(V // _PROJ_RB,),
        in_specs=[
            pl.BlockSpec((_PROJ_RB, E), lambda i: (i, 0)),
            pl.BlockSpec((E, DP), lambda i: (0, 0)),
            pl.BlockSpec((1, DP), lambda i: (0, 0)),
        ],
        out_specs=pl.BlockSpec((_PROJ_RB, DPH), lambda i: (i, 0)),
        out_shape=jax.ShapeDtypeStruct((V, DPH), jnp.uint32),
    )(emb, w1t, b1p)


_sc_mesh = plsc.VectorSubcoreMesh(core_axis_name="c", subcore_axis_name="s")


@functools.partial(
    pl.kernel,
    out_type=jax.ShapeDtypeStruct((NIDXC, DPH), jnp.uint32),
    mesh=_sc_mesh,
    scratch_types=[
        pltpu.VMEM((ROWS_PER_W // GSIZE, GSIZE), jnp.int32),
        pltpu.VMEM((CHUNK, DPH), jnp.uint32),
        pltpu.SemaphoreType.DMA,
    ],
    compiler_params=pltpu.CompilerParams(use_tc_tiling_on_sc=False),
)
def _sc_gather(p_hbm, x_hbm, out_hbm, idx_v, rows_v, sem):
    wid = lax.axis_index("s") * NC + lax.axis_index("c")
    g_per_w = ROWS_PER_W // GSIZE
    base = wid * ROWS_PER_W
    pltpu.sync_copy(x_hbm.at[pl.ds(wid * g_per_w, g_per_w)], idx_v)

    def chunk_body(j, carry):
        copies = []
        for t in range(GPC):
            g = j * GPC + t
            copies.append(
                pltpu.async_copy(
                    p_hbm.at[idx_v.at[g]],
                    rows_v.at[pl.ds(t * GSIZE, GSIZE)],
                    sem,
                )
            )
        for cp in copies:
            cp.wait()
        pltpu.sync_copy(rows_v, out_hbm.at[pl.ds(base + j * CHUNK, CHUNK)])
        return carry

    lax.fori_loop(0, NCHUNK, chunk_body, 0)


_OUT_RB = 1024


def _out_body(g_ref, w2lo_ref, w2hi_ref, b2p_ref, out_ref):
    gp = g_ref[...]
    glo = jax.lax.bitcast_convert_type(
        jax.lax.convert_element_type(gp & jnp.uint32(0xFFFF), jnp.uint16),
        jnp.bfloat16,
    )
    ghi = jax.lax.bitcast_convert_type(
        jax.lax.convert_element_type(gp >> 16, jnp.uint16), jnp.bfloat16
    )
    out_ref[...] = (
        jnp.dot(glo, w2lo_ref[...], preferred_element_type=jnp.float32)
        + jnp.dot(ghi, w2hi_ref[...], preferred_element_type=jnp.float32)
        + b2p_ref[...]
    )


def _final_matmul(g2d, w2lo, w2hi, b2p):
    return pl.pallas_call(
        _out_body,
        grid=(BCH // _OUT_RB,),
        in_specs=[
            pl.BlockSpec((_OUT_RB, S * DPH), lambda i: (i, 0)),
            pl.BlockSpec((S * DPH, CP), lambda i: (0, 0)),
            pl.BlockSpec((S * DPH, CP), lambda i: (0, 0)),
            pl.BlockSpec((1, CP), lambda i: (0, 0)),
        ],
        out_specs=pl.BlockSpec((_OUT_RB, CP), lambda i: (i, 0)),
        out_shape=jax.ShapeDtypeStruct((BCH, CP), jnp.float32),
    )(g2d, w2lo, w2hi, b2p)


def kernel(x, emb, W1, b1, W2, b2):
    w1t = jnp.zeros((E, DP), jnp.float32).at[:, :H].set(W1.T)
    b1p = jnp.zeros((1, DP), jnp.float32).at[0, :H].set(b1)
    w2r = jnp.zeros((C, S, DP), jnp.float32).at[:, :, :H].set(
        W2.reshape(C, S, H)
    )
    w2lo = jnp.zeros((S * DPH, CP), jnp.float32).at[:, :C].set(
        w2r[:, :, :DPH].reshape(C, S * DPH).T
    ).astype(jnp.bfloat16)
    w2hi = jnp.zeros((S * DPH, CP), jnp.float32).at[:, :C].set(
        w2r[:, :, DPH:].reshape(C, S * DPH).T
    ).astype(jnp.bfloat16)
    b2p = jnp.zeros((1, CP), jnp.float32).at[0, :C].set(b2)

    p_pad = _project_table(emb, w1t, b1p)

    x2 = x.astype(jnp.int32).reshape(NIDXC // GSIZE, GSIZE)
    g = _sc_gather(p_pad, x2)
    return _final_matmul(g.reshape(BCH, S * DPH), w2lo, w2hi, b2p)[:, :C]


# speedup vs baseline: 1.1224x; 1.1224x over previous
import functools

import jax
import jax.numpy as jnp
from jax import lax
from jax.experimental import pallas as pl
from jax.experimental.pallas import tpu as pltpu
from jax.experimental.pallas import tpu_sc as plsc

V = 100000
E = 768
H = 50
S = 30
C = 10
B = 16384
DP = 64
CP = 128

NIDX = B * S
NC, NS = 2, 16
NW = NC * NS
GSIZE = 128

NSPLIT = 2
BCH = B // NSPLIT
NIDXC = BCH * S
ROWS_PER_W = NIDXC // NW
GPC = 10
CHUNK = GSIZE * GPC
NCHUNK = ROWS_PER_W // CHUNK
DPH = DP // 2


_PROJ_RB = 2000


def _proj_body(emb_ref, w1t_ref, b1_ref, out_ref):
    y = (
        jnp.dot(emb_ref[...], w1t_ref[...], preferred_element_type=jnp.float32)
        + b1_ref[...]
    ).astype(jnp.bfloat16)
    lo = jax.lax.convert_element_type(
        jax.lax.bitcast_convert_type(y[:, :DPH], jnp.uint16), jnp.uint32
    )
    hi = jax.lax.convert_element_type(
        jax.lax.bitcast_convert_type(y[:, DPH:], jnp.uint16), jnp.uint32
    )
    out_ref[...] = lo | (hi << 16)


def _project_table(emb, w1t, b1p):
    return pl.pallas_call(
        _proj_body,
        grid=(V // _PROJ_RB,),
        in_specs=[
            pl.BlockSpec((_PROJ_RB, E), lambda i: (i, 0)),
            pl.BlockSpec((E, DP), lambda i: (0, 0)),
            pl.BlockSpec((1, DP), lambda i: (0, 0)),
        ],
        out_specs=pl.BlockSpec((_PROJ_RB, DPH), lambda i: (i, 0)),
        out_shape=jax.ShapeDtypeStruct((V, DPH), jnp.uint32),
    )(emb, w1t, b1p)


_sc_mesh = plsc.VectorSubcoreMesh(core_axis_name="c", subcore_axis_name="s")


@functools.partial(
    pl.kernel,
    out_type=jax.ShapeDtypeStruct((NIDXC, DPH), jnp.uint32),
    mesh=_sc_mesh,
    scratch_types=[
        pltpu.VMEM((ROWS_PER_W // GSIZE, GSIZE), jnp.int32),
        pltpu.VMEM((2, CHUNK, DPH), jnp.uint32),
        pltpu.SemaphoreType.DMA,
        pltpu.SemaphoreType.DMA,
        pltpu.SemaphoreType.DMA,
    ],
    compiler_params=pltpu.CompilerParams(use_tc_tiling_on_sc=False),
)
def _sc_gather(p_hbm, x_hbm, out_hbm, idx_v, rows_v, gsem, osem0, osem1):
    wid = lax.axis_index("s") * NC + lax.axis_index("c")
    g_per_w = ROWS_PER_W // GSIZE
    base = wid * ROWS_PER_W
    osems = (osem0, osem1)
    pltpu.sync_copy(x_hbm.at[pl.ds(wid * g_per_w, g_per_w)], idx_v)

    def pair_body(jj, carry):
        for b in range(2):
            j = jj * 2 + b
            buf = rows_v.at[b]
            dst = out_hbm.at[pl.ds(base + j * CHUNK, CHUNK)]

            @pl.when(jj > 0)
            def _():
                pltpu.make_async_copy(buf, dst, osems[b]).wait()

            copies = []
            for t in range(GPC):
                copies.append(
                    pltpu.async_copy(
                        p_hbm.at[idx_v.at[j * GPC + t]],
                        buf.at[pl.ds(t * GSIZE, GSIZE)],
                        gsem,
                    )
                )
            for cp in copies:
                cp.wait()
            pltpu.async_copy(buf, dst, osems[b])
        return carry

    lax.fori_loop(0, NCHUNK // 2, pair_body, 0)
    for b in range(2):
        j = NCHUNK - 2 + b
        pltpu.make_async_copy(
            rows_v.at[b], out_hbm.at[pl.ds(base + j * CHUNK, CHUNK)], osems[b]
        ).wait()


_OUT_RB = 1024


def _out_body(g_ref, w2lo_ref, w2hi_ref, b2p_ref, out_ref):
    gp = g_ref[...]
    glo = jax.lax.bitcast_convert_type(
        jax.lax.convert_element_type(gp & jnp.uint32(0xFFFF), jnp.uint16),
        jnp.bfloat16,
    )
    ghi = jax.lax.bitcast_convert_type(
        jax.lax.convert_element_type(gp >> 16, jnp.uint16), jnp.bfloat16
    )
    out_ref[...] = (
        jnp.dot(glo, w2lo_ref[...], preferred_element_type=jnp.float32)
        + jnp.dot(ghi, w2hi_ref[...], preferred_element_type=jnp.float32)
        + b2p_ref[...]
    )


def _final_matmul(g2d, w2lo, w2hi, b2p):
    return pl.pallas_call(
        _out_body,
        grid=(BCH // _OUT_RB,),
        in_specs=[
            pl.BlockSpec((_OUT_RB, S * DPH), lambda i: (i, 0)),
            pl.BlockSpec((S * DPH, CP), lambda i: (0, 0)),
            pl.BlockSpec((S * DPH, CP), lambda i: (0, 0)),
            pl.BlockSpec((1, CP), lambda i: (0, 0)),
        ],
        out_specs=pl.BlockSpec((_OUT_RB, CP), lambda i: (i, 0)),
        out_shape=jax.ShapeDtypeStruct((BCH, CP), jnp.float32),
    )(g2d, w2lo, w2hi, b2p)


def kernel(x, emb, W1, b1, W2, b2):
    w1t = jnp.zeros((E, DP), jnp.float32).at[:, :H].set(W1.T)
    b1p = jnp.zeros((1, DP), jnp.float32).at[0, :H].set(b1)
    w2r = jnp.zeros((C, S, DP), jnp.float32).at[:, :, :H].set(
        W2.reshape(C, S, H)
    )
    w2lo = jnp.zeros((S * DPH, CP), jnp.float32).at[:, :C].set(
        w2r[:, :, :DPH].reshape(C, S * DPH).T
    ).astype(jnp.bfloat16)
    w2hi = jnp.zeros((S * DPH, CP), jnp.float32).at[:, :C].set(
        w2r[:, :, DPH:].reshape(C, S * DPH).T
    ).astype(jnp.bfloat16)
    b2p = jnp.zeros((1, CP), jnp.float32).at[0, :C].set(b2)

    p_pad = _project_table(emb, w1t, b1p)

    x2 = x.astype(jnp.int32).reshape(NSPLIT, NIDXC // GSIZE, GSIZE)
    outs = []
    for i in range(NSPLIT):
        g = _sc_gather(p_pad, x2[i])
        outs.append(_final_matmul(g.reshape(BCH, S * DPH), w2lo, w2hi, b2p)[:, :C])
    return jnp.concatenate(outs, axis=0)


# speedup vs baseline: 1.1420x; 1.0174x over previous
import functools

import jax
import jax.numpy as jnp
from jax import lax
from jax.experimental import pallas as pl
from jax.experimental.pallas import tpu as pltpu
from jax.experimental.pallas import tpu_sc as plsc

V = 100000
E = 768
H = 50
S = 30
C = 10
B = 16384
DP = 64
CP = 128

NIDX = B * S
NC, NS = 2, 16
NW = NC * NS
GSIZE = 128

NSPLIT = 2
BCH = B // NSPLIT
NIDXC = BCH * S
ROWS_PER_W = NIDXC // NW
GPC = 10
CHUNK = GSIZE * GPC
NCHUNK = ROWS_PER_W // CHUNK
DPH = DP // 2


_PROJ_RB = 4000


def _proj_body(emb_ref, w1t_ref, b1_ref, out_ref):
    y = (
        jnp.dot(emb_ref[...], w1t_ref[...], preferred_element_type=jnp.float32)
        + b1_ref[...]
    ).astype(jnp.bfloat16)
    lo = jax.lax.convert_element_type(
        jax.lax.bitcast_convert_type(y[:, :DPH], jnp.uint16), jnp.uint32
    )
    hi = jax.lax.convert_element_type(
        jax.lax.bitcast_convert_type(y[:, DPH:], jnp.uint16), jnp.uint32
    )
    out_ref[...] = lo | (hi << 16)


def _project_table(emb, w1t, b1p):
    return pl.pallas_call(
        _proj_body,
        grid=(V // _PROJ_RB,),
        in_specs=[
            pl.BlockSpec((_PROJ_RB, E), lambda i: (i, 0)),
            pl.BlockSpec((E, DP), lambda i: (0, 0)),
            pl.BlockSpec((1, DP), lambda i: (0, 0)),
        ],
        out_specs=pl.BlockSpec((_PROJ_RB, DPH), lambda i: (i, 0)),
        out_shape=jax.ShapeDtypeStruct((V, DPH), jnp.uint32),
    )(emb, w1t, b1p)


_sc_mesh = plsc.VectorSubcoreMesh(core_axis_name="c", subcore_axis_name="s")


@functools.partial(
    pl.kernel,
    out_type=jax.ShapeDtypeStruct((NIDXC, DPH), jnp.uint32),
    mesh=_sc_mesh,
    scratch_types=[
        pltpu.VMEM((ROWS_PER_W // GSIZE, GSIZE), jnp.int32),
        pltpu.VMEM((2, CHUNK, DPH), jnp.uint32),
        pltpu.SemaphoreType.DMA,
        pltpu.SemaphoreType.DMA,
        pltpu.SemaphoreType.DMA,
    ],
    compiler_params=pltpu.CompilerParams(use_tc_tiling_on_sc=False),
)
def _sc_gather(p_hbm, x_hbm, out_hbm, idx_v, rows_v, gsem, osem0, osem1):
    wid = lax.axis_index("s") * NC + lax.axis_index("c")
    g_per_w = ROWS_PER_W // GSIZE
    base = wid * ROWS_PER_W
    osems = (osem0, osem1)
    pltpu.sync_copy(x_hbm.at[pl.ds(wid * g_per_w, g_per_w)], idx_v)

    def pair_body(jj, carry):
        for b in range(2):
            j = jj * 2 + b
            buf = rows_v.at[b]
            dst = out_hbm.at[pl.ds(base + j * CHUNK, CHUNK)]

            @pl.when(jj > 0)
            def _():
                pltpu.make_async_copy(buf, dst, osems[b]).wait()

            copies = []
            for t in range(GPC):
                copies.append(
                    pltpu.async_copy(
                        p_hbm.at[idx_v.at[j * GPC + t]],
                        buf.at[pl.ds(t * GSIZE, GSIZE)],
                        gsem,
                    )
                )
            for cp in copies:
                cp.wait()
            pltpu.async_copy(buf, dst, osems[b])
        return carry

    lax.fori_loop(0, NCHUNK // 2, pair_body, 0)
    for b in range(2):
        j = NCHUNK - 2 + b
        pltpu.make_async_copy(
            rows_v.at[b], out_hbm.at[pl.ds(base + j * CHUNK, CHUNK)], osems[b]
        ).wait()


_OUT_RB = 1024


def _out_body(g_ref, w2lo_ref, w2hi_ref, b2p_ref, out_ref):
    gp = g_ref[...]
    glo = jax.lax.bitcast_convert_type(
        jax.lax.convert_element_type(gp & jnp.uint32(0xFFFF), jnp.uint16),
        jnp.bfloat16,
    )
    ghi = jax.lax.bitcast_convert_type(
        jax.lax.convert_element_type(gp >> 16, jnp.uint16), jnp.bfloat16
    )
    acc = (
        jnp.dot(glo, w2lo_ref[...], preferred_element_type=jnp.float32)
        + jnp.dot(ghi, w2hi_ref[...], preferred_element_type=jnp.float32)
        + b2p_ref[...]
    )
    out_ref[...] = acc[:, :C]


def _final_matmul(g2d, w2lo, w2hi, b2p):
    return pl.pallas_call(
        _out_body,
        grid=(BCH // _OUT_RB,),
        in_specs=[
            pl.BlockSpec((_OUT_RB, S * DPH), lambda i: (i, 0)),
            pl.BlockSpec((S * DPH, CP), lambda i: (0, 0)),
            pl.BlockSpec((S * DPH, CP), lambda i: (0, 0)),
            pl.BlockSpec((1, CP), lambda i: (0, 0)),
        ],
        out_specs=pl.BlockSpec((_OUT_RB, C), lambda i: (i, 0)),
        out_shape=jax.ShapeDtypeStruct((BCH, C), jnp.float32),
    )(g2d, w2lo, w2hi, b2p)


def kernel(x, emb, W1, b1, W2, b2):
    w1t = jnp.zeros((E, DP), jnp.float32).at[:, :H].set(W1.T)
    b1p = jnp.zeros((1, DP), jnp.float32).at[0, :H].set(b1)
    w2r = jnp.zeros((C, S, DP), jnp.float32).at[:, :, :H].set(
        W2.reshape(C, S, H)
    )
    w2lo = jnp.zeros((S * DPH, CP), jnp.float32).at[:, :C].set(
        w2r[:, :, :DPH].reshape(C, S * DPH).T
    ).astype(jnp.bfloat16)
    w2hi = jnp.zeros((S * DPH, CP), jnp.float32).at[:, :C].set(
        w2r[:, :, DPH:].reshape(C, S * DPH).T
    ).astype(jnp.bfloat16)
    b2p = jnp.zeros((1, CP), jnp.float32).at[0, :C].set(b2)

    p_pad = _project_table(emb, w1t, b1p)

    x2 = x.astype(jnp.int32).reshape(NSPLIT, NIDXC // GSIZE, GSIZE)
    outs = []
    for i in range(NSPLIT):
        g = _sc_gather(p_pad, x2[i])
        outs.append(_final_matmul(g.reshape(BCH, S * DPH), w2lo, w2hi, b2p))
    return jnp.concatenate(outs, axis=0)
